# Initial kernel scaffold; baseline (speedup 1.0000x reference)
import jax
import jax.numpy as jnp
from jax.experimental import pallas as pl


def kernel(x, Wr, br, W1, b1, W2, b2):
    raise NotImplementedError("write your pallas kernel here")


# speedup vs baseline: 1.0264x; 1.0264x over previous
import functools

import jax
import jax.numpy as jnp
from jax.experimental import pallas as pl
from jax.experimental.pallas import tpu as pltpu

B, S, D, H, E, TOP_K = 2, 2048, 1024, 2048, 8, 2
N = B * S
TBLK = 256
T = N // TBLK


def _moe_dense_body(x_ref, wr_ref, br_ref, w1_ref, b1_ref, w2_ref, b2_ref,
                    out_ref, rt_ref):
    e = pl.program_id(0)
    t = pl.program_id(1)
    rows = pl.ds(t * TBLK, TBLK)

    @pl.when(e == 0)
    def _router():
        x = x_ref[...]
        logits = jnp.dot(x, wr_ref[...], preferred_element_type=jnp.float32)
        logits = logits + br_ref[...]
        iota_e = jax.lax.broadcasted_iota(jnp.int32, (TBLK, E), 1).astype(
            jnp.float32)
        m0 = jnp.max(logits, axis=1, keepdims=True)
        e0 = jnp.min(jnp.where(logits == m0, iota_e, jnp.float32(E)), axis=1,
                     keepdims=True)
        logits2 = jnp.where(iota_e == e0, -jnp.inf, logits)
        m1 = jnp.max(logits2, axis=1, keepdims=True)
        e1 = jnp.min(jnp.where(logits2 == m1, iota_e, jnp.float32(E)), axis=1,
                     keepdims=True)
        w0 = 1.0 / (1.0 + jnp.exp(m1 - m0))
        w1 = 1.0 - w0
        rt_ref[rows, :] = jnp.concatenate([w0, w1, e0, e1], axis=1)

    rt = rt_ref[rows, :]
    ef = jnp.float32(0) + e.astype(jnp.float32)
    coeff = (rt[:, 0:1] * (rt[:, 2:3] == ef).astype(jnp.float32)
             + rt[:, 1:2] * (rt[:, 3:4] == ef).astype(jnp.float32))

    x = x_ref[...]
    h = jnp.dot(x, w1_ref[0], preferred_element_type=jnp.float32) + b1_ref[0]
    h = jax.nn.gelu(h)
    y = jnp.dot(h, w2_ref[0], preferred_element_type=jnp.float32) + b2_ref[0]
    cy = coeff * y

    @pl.when(e == 0)
    def _init():
        out_ref[rows, :] = cy

    @pl.when(e > 0)
    def _acc():
        out_ref[rows, :] = out_ref[rows, :] + cy


@functools.partial(jax.jit, static_argnames=())
def _moe_dense(x2d, Wr, br2, W1, b1, W2, b2):
    return pl.pallas_call(
        _moe_dense_body,
        grid=(E, T),
        in_specs=[
            pl.BlockSpec((TBLK, D), lambda e, t: (t, 0)),
            pl.BlockSpec((D, E), lambda e, t: (0, 0)),
            pl.BlockSpec((1, E), lambda e, t: (0, 0)),
            pl.BlockSpec((1, D, H), lambda e, t: (e, 0, 0)),
            pl.BlockSpec((1, 1, H), lambda e, t: (e, 0, 0)),
            pl.BlockSpec((1, H, D), lambda e, t: (e, 0, 0)),
            pl.BlockSpec((1, 1, D), lambda e, t: (e, 0, 0)),
        ],
        out_specs=pl.BlockSpec((N, D), lambda e, t: (0, 0)),
        out_shape=jax.ShapeDtypeStruct((N, D), jnp.float32),
        scratch_shapes=[
            pltpu.VMEM((N, 4), jnp.float32),
        ],
        compiler_params=pltpu.CompilerParams(
            dimension_semantics=("arbitrary", "arbitrary"),
        ),
    )(x2d, Wr, br2, W1, b1.reshape(E, 1, H), W2, b2.reshape(E, 1, D))


def kernel(x, Wr, br, W1, b1, W2, b2):
    x2d = x.reshape(N, D)
    br2 = br.reshape(1, E)
    out = _moe_dense(x2d, Wr, br2, W1, b1, W2, b2)
    aux_loss = jnp.zeros((), dtype=x.dtype)
    return (out.reshape(B, S, D), aux_loss)


# speedup vs baseline: 1.5223x; 1.4832x over previous
import functools

import jax
import jax.numpy as jnp
from jax import lax
from jax.experimental import pallas as pl
from jax.experimental.pallas import tpu as pltpu
from jax.experimental.pallas import tpu_sc as plsc

B, S, D, H, E, TOP_K = 2, 2048, 1024, 2048, 8, 2
N = B * S
P = N * TOP_K
BLK = 256
NB = P // BLK + E
NR = NB * BLK
CHUNK = 128
C = N // CHUNK

NC, NS = 2, 16
NW = NC * NS
TOK_W = N // NW


def _router_body(x_ref, wr_ref, br_ref, e0_ref, e1_ref, r0_ref, r1_ref,
                 w0b_ref, w1b_ref, cnt_ref, acc_ref):
    c = pl.program_id(0)

    @pl.when(c == 0)
    def _init():
        acc_ref[...] = jnp.zeros((1, E), jnp.float32)

    x = x_ref[...]
    logits = jnp.dot(x, wr_ref[...], preferred_element_type=jnp.float32)
    logits = logits + br_ref[...]
    iota_e = lax.broadcasted_iota(jnp.int32, (CHUNK, E), 1).astype(jnp.float32)
    m0 = jnp.max(logits, axis=1, keepdims=True)
    e0 = jnp.min(jnp.where(logits == m0, iota_e, jnp.float32(E)), axis=1,
                 keepdims=True)
    logits2 = jnp.where(iota_e == e0, -jnp.inf, logits)
    m1 = jnp.max(logits2, axis=1, keepdims=True)
    e1 = jnp.min(jnp.where(logits2 == m1, iota_e, jnp.float32(E)), axis=1,
                 keepdims=True)
    w0 = 1.0 / (1.0 + jnp.exp(m1 - m0))
    w1 = 1.0 - w0

    oh0 = (iota_e == e0).astype(jnp.float32)
    oh1 = (iota_e == e1).astype(jnp.float32)
    sc = oh0 + oh1
    ir = lax.broadcasted_iota(jnp.int32, (CHUNK, CHUNK), 0)
    ic = lax.broadcasted_iota(jnp.int32, (CHUNK, CHUNK), 1)
    ltri = (ic < ir).astype(jnp.float32)
    pref = jnp.dot(ltri, sc, preferred_element_type=jnp.float32)
    pref = pref + acc_ref[...]
    r0 = jnp.sum(pref * oh0, axis=1, keepdims=True)
    r1 = jnp.sum(pref * oh1, axis=1, keepdims=True)

    e0_ref[...] = e0.reshape(1, 1, CHUNK)
    e1_ref[...] = e1.reshape(1, 1, CHUNK)
    r0_ref[...] = r0.reshape(1, 1, CHUNK)
    r1_ref[...] = r1.reshape(1, 1, CHUNK)
    w0b_ref[...] = jnp.broadcast_to(w0, (CHUNK, 16))
    w1b_ref[...] = jnp.broadcast_to(w1, (CHUNK, 16))
    newacc = acc_ref[...] + jnp.sum(sc, axis=0, keepdims=True)
    acc_ref[...] = newacc
    cnt_ref[...] = newacc


def _router(x2d, Wr, br2):
    shp = jax.ShapeDtypeStruct((C, 1, CHUNK), jnp.float32)
    return pl.pallas_call(
        _router_body,
        grid=(C,),
        in_specs=[
            pl.BlockSpec((CHUNK, D), lambda c: (c, 0)),
            pl.BlockSpec((D, E), lambda c: (0, 0)),
            pl.BlockSpec((1, E), lambda c: (0, 0)),
        ],
        out_specs=[pl.BlockSpec((1, 1, CHUNK), lambda c: (c, 0, 0))] * 4
        + [pl.BlockSpec((CHUNK, 16), lambda c: (c, 0))] * 2
        + [pl.BlockSpec((1, E), lambda c: (0, 0))],
        out_shape=[shp] * 4
        + [jax.ShapeDtypeStruct((N, 16), jnp.float32)] * 2
        + [jax.ShapeDtypeStruct((1, E), jnp.float32)],
        scratch_shapes=[pltpu.VMEM((1, E), jnp.float32)],
        compiler_params=pltpu.CompilerParams(
            dimension_semantics=("arbitrary",)),
    )(x2d, Wr, br2)


def _pos_body(cnt_ref, e0_ref, e1_ref, r0_ref, r1_ref,
              pos0_ref, pos1_ref, meta_ref):
    cnt = cnt_ref[...]
    pcnt = jnp.floor((cnt + (BLK - 1)) * (1.0 / BLK)) * BLK
    ie = lax.broadcasted_iota(jnp.int32, (E, E), 0).astype(jnp.float32)
    je = lax.broadcasted_iota(jnp.int32, (E, E), 1).astype(jnp.float32)
    excl = (ie < je).astype(jnp.float32)
    pstart = jnp.dot(pcnt, excl, preferred_element_type=jnp.float32)

    e0 = e0_ref[...].reshape(C, CHUNK)
    e1 = e1_ref[...].reshape(C, CHUNK)
    sel0 = jnp.zeros((C, CHUNK), jnp.float32)
    sel1 = jnp.zeros((C, CHUNK), jnp.float32)
    for e in range(E):
        ps_e = pstart[0:1, e:e + 1]
        sel0 = sel0 + jnp.where(e0 == jnp.float32(e), ps_e, 0.0)
        sel1 = sel1 + jnp.where(e1 == jnp.float32(e), ps_e, 0.0)
    pos0 = sel0 + r0_ref[...].reshape(C, CHUNK)
    pos1 = sel1 + r1_ref[...].reshape(C, CHUNK)
    pos0_ref[...] = pos0.astype(jnp.int32).reshape(C, 1, CHUNK)
    pos1_ref[...] = pos1.astype(jnp.int32).reshape(C, 1, CHUNK)

    bl = lax.broadcasted_iota(jnp.int32, (1, 64), 1).astype(jnp.float32) * BLK
    nexp = jnp.zeros((1, 64), jnp.float32)
    for e in range(E):
        nexp = nexp + (bl >= pstart[0:1, e:e + 1]).astype(jnp.float32)
    blkexp = jnp.clip(nexp - 1.0, 0.0, jnp.float32(E - 1))
    total = pstart[0:1, E - 1:E] + pcnt[0:1, E - 1:E]
    nreal = total * (1.0 / BLK)
    icol = lax.broadcasted_iota(jnp.int32, (1, 64), 1)
    meta = jnp.where(icol == 40, nreal, blkexp)
    meta_ref[...] = meta.astype(jnp.int32)


def _positions(cnt, e0, e1, r0, r1):
    shp = jax.ShapeDtypeStruct((C, 1, CHUNK), jnp.int32)
    return pl.pallas_call(
        _pos_body,
        out_shape=[shp, shp, jax.ShapeDtypeStruct((1, 64), jnp.int32)],
    )(cnt, e0, e1, r0, r1)


_XCH = 32


def _scatter_body(x_hbm, p0_hbm, p1_hbm, xs_hbm, i0_v, i1_v, xb_v,
                  sem0, sem1):
    wid = lax.axis_index("s") * NC + lax.axis_index("c")
    pltpu.sync_copy(p0_hbm.at[wid], i0_v)
    pltpu.sync_copy(p1_hbm.at[wid], i1_v)
    for c in range(TOK_W // _XCH):
        base = wid * TOK_W + c * _XCH
        pltpu.sync_copy(x_hbm.at[pl.ds(base, _XCH)], xb_v)
        cp0 = pltpu.async_copy(xb_v, xs_hbm.at[i0_v.at[c]], sem0)
        cp1 = pltpu.async_copy(xb_v, xs_hbm.at[i1_v.at[c]], sem1)
        cp0.wait()
        cp1.wait()


@functools.cache
def _sc_scatter_kernel():
    return pl.kernel(
        _scatter_body,
        out_type=jax.ShapeDtypeStruct((NR, D), jnp.float32),
        mesh=plsc.VectorSubcoreMesh(core_axis_name="c", subcore_axis_name="s"),
        scratch_types=[
            pltpu.VMEM((TOK_W // _XCH, _XCH), jnp.int32),
            pltpu.VMEM((TOK_W // _XCH, _XCH), jnp.int32),
            pltpu.VMEM((_XCH, D), jnp.float32),
            pltpu.SemaphoreType.DMA,
            pltpu.SemaphoreType.DMA,
        ],
    )


def _sc_scatter(x2d, p0s, p1s):
    return _sc_scatter_kernel()(x2d, p0s, p1s)


def _ffn_body(meta_ref, xs_ref, w1_ref, b1_ref, w2_ref, b2_ref, ys_ref):
    b = pl.program_id(0)

    @pl.when(b < meta_ref[40])
    def _compute():
        xb = xs_ref[...]
        h = jnp.dot(xb, w1_ref[0], preferred_element_type=jnp.float32)
        h = jax.nn.gelu(h + b1_ref[0])
        y = jnp.dot(h, w2_ref[0], preferred_element_type=jnp.float32)
        ys_ref[...] = y + b2_ref[0]


def _grouped_ffn(meta1d, xs, W1, b1, W2, b2):
    grid_spec = pltpu.PrefetchScalarGridSpec(
        num_scalar_prefetch=1,
        grid=(NB,),
        in_specs=[
            pl.BlockSpec((BLK, D), lambda b, m: (b, 0)),
            pl.BlockSpec((1, D, H), lambda b, m: (m[b], 0, 0)),
            pl.BlockSpec((1, 1, H), lambda b, m: (m[b], 0, 0)),
            pl.BlockSpec((1, H, D), lambda b, m: (m[b], 0, 0)),
            pl.BlockSpec((1, 1, D), lambda b, m: (m[b], 0, 0)),
        ],
        out_specs=pl.BlockSpec((BLK, D), lambda b, m: (b, 0)),
    )
    return pl.pallas_call(
        _ffn_body,
        grid_spec=grid_spec,
        out_shape=jax.ShapeDtypeStruct((NR, D), jnp.float32),
        compiler_params=pltpu.CompilerParams(
            dimension_semantics=("arbitrary",)),
    )(meta1d, xs, W1, b1.reshape(E, 1, H), W2, b2.reshape(E, 1, D))


_GCH = 16


def _combine_body(ys_hbm, p0_hbm, p1_hbm, w0b_hbm, w1b_hbm, out_hbm,
                  i0_v, i1_v, wv0_v, wv1_v, a_v, b_v, o_v, sem0, sem1):
    wid = lax.axis_index("s") * NC + lax.axis_index("c")
    pltpu.sync_copy(p0_hbm.at[wid], i0_v)
    pltpu.sync_copy(p1_hbm.at[wid], i1_v)
    for c in range(TOK_W // _GCH):
        tokbase = wid * TOK_W + c * _GCH
        ga = pltpu.async_copy(ys_hbm.at[i0_v.at[c]], a_v, sem0)
        gb = pltpu.async_copy(ys_hbm.at[i1_v.at[c]], b_v, sem1)
        pltpu.sync_copy(w0b_hbm.at[pl.ds(tokbase, _GCH)], wv0_v)
        pltpu.sync_copy(w1b_hbm.at[pl.ds(tokbase, _GCH)], wv1_v)
        ga.wait()
        gb.wait()
        for t in range(_GCH):
            w0s = wv0_v[t]
            w1s = wv1_v[t]

            def _row(d, _, t=t, w0s=w0s, w1s=w1s):
                sl = pl.ds(d * 16, 16)
                o_v[t, sl] = w0s * a_v[t, sl] + w1s * b_v[t, sl]
                return _

            lax.fori_loop(0, D // 16, _row, 0)
        pltpu.sync_copy(o_v, out_hbm.at[pl.ds(tokbase, _GCH)])


@functools.cache
def _sc_combine_kernel():
    return pl.kernel(
        _combine_body,
        out_type=jax.ShapeDtypeStruct((N, D), jnp.float32),
        mesh=plsc.VectorSubcoreMesh(core_axis_name="c", subcore_axis_name="s"),
        scratch_types=[
            pltpu.VMEM((TOK_W // _GCH, _GCH), jnp.int32),
            pltpu.VMEM((TOK_W // _GCH, _GCH), jnp.int32),
            pltpu.VMEM((_GCH, 16), jnp.float32),
            pltpu.VMEM((_GCH, 16), jnp.float32),
            pltpu.VMEM((_GCH, D), jnp.float32),
            pltpu.VMEM((_GCH, D), jnp.float32),
            pltpu.VMEM((_GCH, D), jnp.float32),
            pltpu.SemaphoreType.DMA,
            pltpu.SemaphoreType.DMA,
        ],
    )


def _sc_combine(ys, p0g, p1g, w0b, w1b):
    return _sc_combine_kernel()(ys, p0g, p1g, w0b, w1b)


def kernel(x, Wr, br, W1, b1, W2, b2):
    x2d = x.reshape(N, D)
    br2 = br.reshape(1, E)
    e0, e1, r0, r1, w0b, w1b, cnt = _router(x2d, Wr, br2)
    pos0, pos1, meta = _positions(cnt, e0, e1, r0, r1)
    p0s = pos0.reshape(NW, TOK_W // _XCH, _XCH)
    p1s = pos1.reshape(NW, TOK_W // _XCH, _XCH)
    xs = _sc_scatter(x2d, p0s, p1s)
    ys = _grouped_ffn(meta.reshape(64), xs, W1, b1, W2, b2)
    p0g = pos0.reshape(NW, TOK_W // _GCH, _GCH)
    p1g = pos1.reshape(NW, TOK_W // _GCH, _GCH)
    out = _sc_combine(ys, p0g, p1g, w0b, w1b)
    aux_loss = jnp.zeros((), dtype=x.dtype)
    return (out.reshape(B, S, D), aux_loss)


# speedup vs baseline: 1.8286x; 1.2012x over previous
import functools

import jax
import jax.numpy as jnp
from jax import lax
from jax.experimental import pallas as pl
from jax.experimental.pallas import tpu as pltpu
from jax.experimental.pallas import tpu_sc as plsc

B, S, D, H, E, TOP_K = 2, 2048, 1024, 2048, 8, 2
N = B * S
P = N * TOP_K
BLK = 256
NB = P // BLK + E
NR = NB * BLK
CHUNK = 512
C = N // CHUNK

NC, NS = 2, 16
NW = NC * NS
TOK_W = N // NW


def _router_body(x_ref, wr_ref, br_ref, e0_ref, e1_ref, w0b_ref, w1b_ref,
                 pos0_ref, pos1_ref, meta_ref, acc_ref, e0s, e1s, r0s, r1s):
    c = pl.program_id(0)

    @pl.when(c == 0)
    def _init():
        acc_ref[...] = jnp.zeros((1, E), jnp.float32)

    @pl.when(c < C)
    def _route():
        x = x_ref[...]
        logits = jnp.dot(x, wr_ref[...], preferred_element_type=jnp.float32)
        logits = logits + br_ref[...]
        iota_e = lax.broadcasted_iota(jnp.int32, (CHUNK, E), 1).astype(
            jnp.float32)
        m0 = jnp.max(logits, axis=1, keepdims=True)
        e0 = jnp.min(jnp.where(logits == m0, iota_e, jnp.float32(E)), axis=1,
                     keepdims=True)
        logits2 = jnp.where(iota_e == e0, -jnp.inf, logits)
        m1 = jnp.max(logits2, axis=1, keepdims=True)
        e1 = jnp.min(jnp.where(logits2 == m1, iota_e, jnp.float32(E)), axis=1,
                     keepdims=True)
        w0 = 1.0 / (1.0 + jnp.exp(m1 - m0))
        w1 = 1.0 - w0

        oh0 = (iota_e == e0).astype(jnp.float32)
        oh1 = (iota_e == e1).astype(jnp.float32)
        sc = oh0 + oh1
        ir = lax.broadcasted_iota(jnp.int32, (CHUNK, CHUNK), 0)
        ic = lax.broadcasted_iota(jnp.int32, (CHUNK, CHUNK), 1)
        ltri = (ic < ir).astype(jnp.float32)
        pref = jnp.dot(ltri, sc, preferred_element_type=jnp.float32)
        pref = pref + acc_ref[...]
        r0 = jnp.sum(pref * oh0, axis=1, keepdims=True)
        r1 = jnp.sum(pref * oh1, axis=1, keepdims=True)

        e0_ref[...] = e0.reshape(1, 1, CHUNK)
        e1_ref[...] = e1.reshape(1, 1, CHUNK)
        w0b_ref[...] = jnp.broadcast_to(w0, (CHUNK, 16))
        w1b_ref[...] = jnp.broadcast_to(w1, (CHUNK, 16))
        e0s[pl.ds(c, 1), :] = e0.reshape(1, CHUNK)
        e1s[pl.ds(c, 1), :] = e1.reshape(1, CHUNK)
        r0s[pl.ds(c, 1), :] = r0.reshape(1, CHUNK)
        r1s[pl.ds(c, 1), :] = r1.reshape(1, CHUNK)
        acc_ref[...] = acc_ref[...] + jnp.sum(sc, axis=0, keepdims=True)

    @pl.when(c == C)
    def _positions():
        cnt = acc_ref[...]
        pcnt = jnp.floor((cnt + (BLK - 1)) * (1.0 / BLK)) * BLK
        ie = lax.broadcasted_iota(jnp.int32, (E, E), 0).astype(jnp.float32)
        je = lax.broadcasted_iota(jnp.int32, (E, E), 1).astype(jnp.float32)
        excl = (ie < je).astype(jnp.float32)
        pstart = jnp.dot(pcnt, excl,
                         preferred_element_type=jnp.float32)

        e0 = e0s[...]
        e1 = e1s[...]
        sel0 = jnp.zeros((C, CHUNK), jnp.float32)
        sel1 = jnp.zeros((C, CHUNK), jnp.float32)
        for e in range(E):
            ps_e = pstart[0:1, e:e + 1]
            sel0 = sel0 + jnp.where(e0 == jnp.float32(e), ps_e, 0.0)
            sel1 = sel1 + jnp.where(e1 == jnp.float32(e), ps_e, 0.0)
        pos0 = sel0 + r0s[...]
        pos1 = sel1 + r1s[...]
        pos0_ref[...] = pos0.astype(jnp.int32).reshape(C, 1, CHUNK)
        pos1_ref[...] = pos1.astype(jnp.int32).reshape(C, 1, CHUNK)

        bl = lax.broadcasted_iota(jnp.int32, (1, 64), 1).astype(
            jnp.float32) * BLK
        nexp = jnp.zeros((1, 64), jnp.float32)
        for e in range(E):
            nexp = nexp + (bl >= pstart[0:1, e:e + 1]).astype(jnp.float32)
        blkexp = jnp.clip(nexp - 1.0, 0.0, jnp.float32(E - 1))
        total = pstart[0:1, E - 1:E] + pcnt[0:1, E - 1:E]
        nreal = total * (1.0 / BLK)
        icol = lax.broadcasted_iota(jnp.int32, (1, 64), 1)
        meta = jnp.where(icol == 40, nreal, blkexp)
        meta_ref[...] = meta.astype(jnp.int32)


def _router(x2d, Wr, br2):
    shp3 = jax.ShapeDtypeStruct((C, 1, CHUNK), jnp.float32)
    cmin = lambda c: jnp.minimum(c, C - 1)
    return pl.pallas_call(
        _router_body,
        grid=(C + 1,),
        in_specs=[
            pl.BlockSpec((CHUNK, D), lambda c: (cmin(c), 0)),
            pl.BlockSpec((D, E), lambda c: (0, 0)),
            pl.BlockSpec((1, E), lambda c: (0, 0)),
        ],
        out_specs=[pl.BlockSpec((1, 1, CHUNK), lambda c: (cmin(c), 0, 0))] * 2
        + [pl.BlockSpec((CHUNK, 16), lambda c: (cmin(c), 0))] * 2
        + [pl.BlockSpec((C, 1, CHUNK), lambda c: (0, 0, 0))] * 2
        + [pl.BlockSpec((1, 64), lambda c: (0, 0))],
        out_shape=[shp3] * 2
        + [jax.ShapeDtypeStruct((N, 16), jnp.float32)] * 2
        + [jax.ShapeDtypeStruct((C, 1, CHUNK), jnp.int32)] * 2
        + [jax.ShapeDtypeStruct((1, 64), jnp.int32)],
        scratch_shapes=[pltpu.VMEM((1, E), jnp.float32)]
        + [pltpu.VMEM((C, CHUNK), jnp.float32)] * 4,
        compiler_params=pltpu.CompilerParams(
            dimension_semantics=("arbitrary",)),
    )(x2d, Wr, br2)


_XCH = 32
_NXC = TOK_W // _XCH


def _scatter_body(x_hbm, p0_hbm, p1_hbm, xs_hbm, i0_v, i1_v, xb_v,
                  semx0, semx1, sem00, sem01, sem10, sem11):
    wid = lax.axis_index("s") * NC + lax.axis_index("c")
    pltpu.sync_copy(p0_hbm.at[wid], i0_v)
    pltpu.sync_copy(p1_hbm.at[wid], i1_v)
    semx = [semx0, semx1]
    sem0 = [sem00, sem01]
    sem1 = [sem10, sem11]
    loads = [None] * _NXC
    scat = [None] * _NXC

    def _issue_load(c):
        base = wid * TOK_W + c * _XCH
        loads[c] = pltpu.async_copy(x_hbm.at[pl.ds(base, _XCH)],
                                    xb_v.at[c % 2], semx[c % 2])

    _issue_load(0)
    for c in range(_NXC):
        buf = c % 2
        if c + 1 < _NXC:
            if c - 1 >= 0:
                scat[c - 1][0].wait()
                scat[c - 1][1].wait()
            _issue_load(c + 1)
        loads[c].wait()
        s0 = pltpu.async_copy(xb_v.at[buf], xs_hbm.at[i0_v.at[c]], sem0[buf])
        s1 = pltpu.async_copy(xb_v.at[buf], xs_hbm.at[i1_v.at[c]], sem1[buf])
        scat[c] = (s0, s1)
    scat[_NXC - 2][0].wait()
    scat[_NXC - 2][1].wait()
    scat[_NXC - 1][0].wait()
    scat[_NXC - 1][1].wait()


@functools.cache
def _sc_scatter_kernel():
    return pl.kernel(
        _scatter_body,
        out_type=jax.ShapeDtypeStruct((NR, D), jnp.float32),
        mesh=plsc.VectorSubcoreMesh(core_axis_name="c", subcore_axis_name="s"),
        scratch_types=[
            pltpu.VMEM((_NXC, _XCH), jnp.int32),
            pltpu.VMEM((_NXC, _XCH), jnp.int32),
            pltpu.VMEM((2, _XCH, D), jnp.float32),
        ] + [pltpu.SemaphoreType.DMA] * 6,
    )


def _sc_scatter(x2d, p0s, p1s):
    return _sc_scatter_kernel()(x2d, p0s, p1s)


def _ffn_body(meta_ref, xs_ref, w1_ref, b1_ref, w2_ref, b2_ref, ys_ref):
    b = pl.program_id(0)

    @pl.when(b < meta_ref[40])
    def _compute():
        xb = xs_ref[...]
        h = jnp.dot(xb, w1_ref[0], preferred_element_type=jnp.float32)
        h = jax.nn.gelu(h + b1_ref[0])
        y = jnp.dot(h, w2_ref[0], preferred_element_type=jnp.float32)
        ys_ref[...] = y + b2_ref[0]


def _grouped_ffn(meta1d, xs, W1, b1, W2, b2):
    grid_spec = pltpu.PrefetchScalarGridSpec(
        num_scalar_prefetch=1,
        grid=(NB,),
        in_specs=[
            pl.BlockSpec((BLK, D), lambda b, m: (b, 0)),
            pl.BlockSpec((1, D, H), lambda b, m: (m[b], 0, 0)),
            pl.BlockSpec((1, 1, H), lambda b, m: (m[b], 0, 0)),
            pl.BlockSpec((1, H, D), lambda b, m: (m[b], 0, 0)),
            pl.BlockSpec((1, 1, D), lambda b, m: (m[b], 0, 0)),
        ],
        out_specs=pl.BlockSpec((BLK, D), lambda b, m: (b, 0)),
    )
    return pl.pallas_call(
        _ffn_body,
        grid_spec=grid_spec,
        out_shape=jax.ShapeDtypeStruct((NR, D), jnp.float32),
        compiler_params=pltpu.CompilerParams(
            dimension_semantics=("arbitrary",)),
    )(meta1d, xs, W1, b1.reshape(E, 1, H), W2, b2.reshape(E, 1, D))


_GCH = 8
_NGC = TOK_W // _GCH


def _combine_body(ys_hbm, p0_hbm, p1_hbm, w0b_hbm, w1b_hbm, out_hbm,
                  i0_v, i1_v, wv0_v, wv1_v, a_v, b_v, o_v,
                  sa0, sa1, sb0, sb1, so0, so1):
    wid = lax.axis_index("s") * NC + lax.axis_index("c")
    tok0 = wid * TOK_W
    pltpu.sync_copy(p0_hbm.at[wid], i0_v)
    pltpu.sync_copy(p1_hbm.at[wid], i1_v)
    pltpu.sync_copy(w0b_hbm.at[pl.ds(tok0, TOK_W)], wv0_v)
    pltpu.sync_copy(w1b_hbm.at[pl.ds(tok0, TOK_W)], wv1_v)
    sa = [sa0, sa1]
    sb = [sb0, sb1]
    so = [so0, so1]
    gath = [None] * _NGC
    outs = [None] * _NGC

    def _issue_gather(c):
        ga = pltpu.async_copy(ys_hbm.at[i0_v.at[c]], a_v.at[c % 2], sa[c % 2])
        gb = pltpu.async_copy(ys_hbm.at[i1_v.at[c]], b_v.at[c % 2], sb[c % 2])
        gath[c] = (ga, gb)

    _issue_gather(0)
    for c in range(_NGC):
        buf = c % 2
        if c + 1 < _NGC:
            _issue_gather(c + 1)
        gath[c][0].wait()
        gath[c][1].wait()
        if c - 2 >= 0:
            outs[c - 2].wait()

        af = a_v.at[buf]
        bf = b_v.at[buf]
        of = o_v.at[buf]

        def _tok(t, _, af=af, bf=bf, of=of, c=c):
            w0s = wv0_v[c * _GCH + t]
            w1s = wv1_v[c * _GCH + t]

            def _grp(j, _):
                for k in range(4):
                    sl = pl.ds(j * 64 + k * 16, 16)
                    of[t, sl] = w0s * af[t, sl] + w1s * bf[t, sl]
                return _

            lax.fori_loop(0, D // 64, _grp, 0)
            return _

        lax.fori_loop(0, _GCH, _tok, 0)
        oc = pltpu.async_copy(o_v.at[buf],
                              out_hbm.at[pl.ds(tok0 + c * _GCH, _GCH)],
                              so[buf])
        outs[c] = oc
    outs[_NGC - 2].wait()
    outs[_NGC - 1].wait()


@functools.cache
def _sc_combine_kernel():
    return pl.kernel(
        _combine_body,
        out_type=jax.ShapeDtypeStruct((N, D), jnp.float32),
        mesh=plsc.VectorSubcoreMesh(core_axis_name="c", subcore_axis_name="s"),
        scratch_types=[
            pltpu.VMEM((_NGC, _GCH), jnp.int32),
            pltpu.VMEM((_NGC, _GCH), jnp.int32),
            pltpu.VMEM((TOK_W, 16), jnp.float32),
            pltpu.VMEM((TOK_W, 16), jnp.float32),
            pltpu.VMEM((2, _GCH, D), jnp.float32),
            pltpu.VMEM((2, _GCH, D), jnp.float32),
            pltpu.VMEM((2, _GCH, D), jnp.float32),
        ] + [pltpu.SemaphoreType.DMA] * 6,
    )


def _sc_combine(ys, p0g, p1g, w0b, w1b):
    return _sc_combine_kernel()(ys, p0g, p1g, w0b, w1b)


def kernel(x, Wr, br, W1, b1, W2, b2):
    x2d = x.reshape(N, D)
    br2 = br.reshape(1, E)
    e0, e1, w0b, w1b, pos0, pos1, meta = _router(x2d, Wr, br2)
    p0s = pos0.reshape(NW, _NXC, _XCH)
    p1s = pos1.reshape(NW, _NXC, _XCH)
    xs = _sc_scatter(x2d, p0s, p1s)
    ys = _grouped_ffn(meta.reshape(64), xs, W1, b1, W2, b2)
    p0g = pos0.reshape(NW, _NGC, _GCH)
    p1g = pos1.reshape(NW, _NGC, _GCH)
    out = _sc_combine(ys, p0g, p1g, w0b, w1b)
    aux_loss = jnp.zeros((), dtype=x.dtype)
    return (out.reshape(B, S, D), aux_loss)
